# Initial kernel scaffold; baseline (speedup 1.0000x reference)
import jax
import jax.numpy as jnp
from jax.experimental import pallas as pl


def kernel(input_ids, emb_table, W, b):
    raise NotImplementedError("write your pallas kernel here")


# speedup vs baseline: 7.7168x; 7.7168x over previous
import functools

import jax
import jax.numpy as jnp
from jax import lax
from jax.experimental import pallas as pl
from jax.experimental.pallas import tpu as pltpu
from jax.experimental.pallas import tpu_sc as plsc

B = 16384
L = 12
VOCAB = 10000
D = 64

_INFO = plsc.get_sparse_core_info()
_NC = _INFO.num_cores
_NS = _INFO.num_subcores
_NW = _NC * _NS
_SEQ_PER_W = B // _NW
_CHUNK_SEQ = 32
_CHUNK_IDX = _CHUNK_SEQ * L
_N_CHUNKS = _SEQ_PER_W // _CHUNK_SEQ
_IDX_ROWS = _SEQ_PER_W * L // 128
_STREAMS_PER_CHUNK = _CHUNK_IDX // 128


def _table_body(e_ref, w_ref, b_ref, out_ref):
    prod = lax.dot_general(
        e_ref[...], w_ref[...],
        dimension_numbers=(((1,), (1,)), ((), ())),
        preferred_element_type=jnp.float32,
    )
    out_ref[...] = (prod + b_ref[...]) * (1.0 / L)


def _transform_table(emb_table, W, b):
    return pl.pallas_call(
        _table_body,
        out_shape=jax.ShapeDtypeStruct((VOCAB, D), jnp.float32),
    )(emb_table, W, b.reshape(1, D))


def _sc_body(table_hbm, idx_hbm, out_hbm, idx_v, rows_v, out_v, sem):
    wid = lax.axis_index("s") * _NC + lax.axis_index("c")

    pltpu.sync_copy(idx_hbm.at[wid], idx_v)

    def chunk_body(c):
        handles = [
            pltpu.async_copy(
                table_hbm.at[idx_v.at[c * _STREAMS_PER_CHUNK + j]],
                rows_v.at[pl.ds(j * 128, 128)],
                sem,
            )
            for j in range(_STREAMS_PER_CHUNK)
        ]
        for h in handles:
            h.wait()

        def seq_body(s, _):
            base = s * L
            for q in range(D // 16):
                acc = rows_v[base, pl.ds(q * 16, 16)]
                for l in range(1, L):
                    acc = acc + rows_v[base + l, pl.ds(q * 16, 16)]
                out_v[c * _CHUNK_SEQ + s, pl.ds(q * 16, 16)] = acc
            return 0

        lax.fori_loop(0, _CHUNK_SEQ, seq_body, 0, unroll=False)

    pl.loop(0, _N_CHUNKS)(chunk_body)

    pltpu.sync_copy(out_v, out_hbm.at[pl.ds(wid * _SEQ_PER_W, _SEQ_PER_W)])


@functools.partial(
    pl.kernel,
    out_type=jax.ShapeDtypeStruct((B, D), jnp.float32),
    mesh=plsc.VectorSubcoreMesh(core_axis_name="c", subcore_axis_name="s"),
    compiler_params=pltpu.CompilerParams(use_tc_tiling_on_sc=False),
    scratch_types=[
        pltpu.VMEM((_IDX_ROWS, 128), jnp.int32),
        pltpu.VMEM((_CHUNK_IDX, D), jnp.float32),
        pltpu.VMEM((_SEQ_PER_W, D), jnp.float32),
        pltpu.SemaphoreType.DMA,
    ],
)
def _sc_gather_pool(table_hbm, idx_hbm, out_hbm, idx_v, rows_v, out_v, sem):
    _sc_body(table_hbm, idx_hbm, out_hbm, idx_v, rows_v, out_v, sem)


@jax.jit
def kernel(input_ids, emb_table, W, b):
    table2 = _transform_table(emb_table, W, b)
    idx = input_ids.astype(jnp.int32).reshape(_NW, _IDX_ROWS, 128)
    return _sc_gather_pool(table2, idx)


# speedup vs baseline: 9.5832x; 1.2419x over previous
import functools

import jax
import jax.numpy as jnp
from jax import lax
from jax.experimental import pallas as pl
from jax.experimental.pallas import tpu as pltpu
from jax.experimental.pallas import tpu_sc as plsc

B = 16384
L = 12
VOCAB = 10000
D = 64

_INFO = plsc.get_sparse_core_info()
_NC = _INFO.num_cores
_NS = _INFO.num_subcores
_NW = _NC * _NS
_SEQ_PER_W = B // _NW
_IDX_PER_W = _SEQ_PER_W * L
_CHUNK_SEQ = 32
_CHUNK_IDX = _CHUNK_SEQ * L
_N_CHUNKS = _SEQ_PER_W // _CHUNK_SEQ
_STREAMS = _CHUNK_IDX // 128


def _table_body(e_ref, w_ref, b_ref, out_ref):
    prod = lax.dot_general(
        e_ref[...], w_ref[...],
        dimension_numbers=(((1,), (1,)), ((), ())),
        preferred_element_type=jnp.float32,
    )
    out_ref[...] = (prod + b_ref[...]) * (1.0 / L)


def _transform_table(emb_table, W, b):
    return pl.pallas_call(
        _table_body,
        out_shape=jax.ShapeDtypeStruct((VOCAB, D), jnp.float32),
    )(emb_table, W, b.reshape(1, D))


def _sc_body(table_hbm, idx_hbm, out_hbm, idx_v, rows0, rows1, out_v,
             sem0, sem1):
    wid = lax.axis_index("s") * _NC + lax.axis_index("c")
    base_idx = wid * _IDX_PER_W

    pltpu.sync_copy(idx_hbm.at[pl.ds(base_idx, _IDX_PER_W)], idx_v)

    bufs = (rows0, rows1)
    sems = (sem0, sem1)

    def copies(c, k):
        buf, sem = bufs[k], sems[k]
        return [
            pltpu.make_async_copy(
                table_hbm.at[idx_v.at[pl.ds(c * _CHUNK_IDX + j * 128, 128)]],
                buf.at[pl.ds(j * 128, 128)],
                sem,
            )
            for j in range(_STREAMS)
        ]

    def fire(c, k):
        for h in copies(c, k):
            h.start()

    def drain(c, k):
        for h in copies(c, k):
            h.wait()

    def compute(c, k):
        buf = bufs[k]

        def seq_body(s, _):
            base = s * L
            for q in range(D // 16):
                acc = buf[base, pl.ds(q * 16, 16)]
                for l in range(1, L):
                    acc = acc + buf[base + l, pl.ds(q * 16, 16)]
                out_v[c * _CHUNK_SEQ + s, pl.ds(q * 16, 16)] = acc
            return 0

        lax.fori_loop(0, _CHUNK_SEQ, seq_body, 0, unroll=2)

    fire(0, 0)

    def pair_body(p):
        c0 = p * 2
        c1 = c0 + 1
        fire(c1, 1)
        drain(c0, 0)
        compute(c0, 0)

        @pl.when(c1 + 1 < _N_CHUNKS)
        def _():
            fire(c1 + 1, 0)

        drain(c1, 1)
        compute(c1, 1)

    pl.loop(0, _N_CHUNKS // 2)(pair_body)

    pltpu.sync_copy(out_v, out_hbm.at[pl.ds(wid * _SEQ_PER_W, _SEQ_PER_W)])


@functools.partial(
    pl.kernel,
    out_type=jax.ShapeDtypeStruct((B, D), jnp.float32),
    mesh=plsc.VectorSubcoreMesh(core_axis_name="c", subcore_axis_name="s"),
    compiler_params=pltpu.CompilerParams(use_tc_tiling_on_sc=False),
    scratch_types=[
        pltpu.VMEM((_IDX_PER_W,), jnp.int32),
        pltpu.VMEM((_CHUNK_IDX, D), jnp.float32),
        pltpu.VMEM((_CHUNK_IDX, D), jnp.float32),
        pltpu.VMEM((_SEQ_PER_W, D), jnp.float32),
        pltpu.SemaphoreType.DMA,
        pltpu.SemaphoreType.DMA,
    ],
)
def _sc_gather_pool(table_hbm, idx_hbm, out_hbm, idx_v, rows0, rows1, out_v,
                    sem0, sem1):
    _sc_body(table_hbm, idx_hbm, out_hbm, idx_v, rows0, rows1, out_v,
             sem0, sem1)


@jax.jit
def kernel(input_ids, emb_table, W, b):
    table2 = _transform_table(emb_table, W, b)
    idx = input_ids.astype(jnp.int32).reshape(-1)
    return _sc_gather_pool(table2, idx)


# speedup vs baseline: 11.3477x; 1.1841x over previous
import functools

import jax
import jax.numpy as jnp
import numpy as np
from jax import lax
from jax.experimental import pallas as pl
from jax.experimental.pallas import tpu as pltpu
from jax.experimental.pallas import tpu_sc as plsc

B = 16384
L = 12
VOCAB = 10000
D = 64

_INFO = plsc.get_sparse_core_info()
_NC = _INFO.num_cores
_NS = _INFO.num_subcores
_NW = _NC * _NS
_SEQ_PER_W = B // _NW
_IDX_PER_W = _SEQ_PER_W * L
_CHUNK_SEQ = 32
_CHUNK_IDX = _CHUNK_SEQ * L
_N_CHUNKS = _SEQ_PER_W // _CHUNK_SEQ
_STREAMS = _CHUNK_IDX // 128

_PERM = np.empty((D,), dtype=np.int32)
for _h in range(D // 32):
    for _i in range(16):
        _PERM[32 * _h + 2 * _i] = 32 * _h + _i
        _PERM[32 * _h + 2 * _i + 1] = 32 * _h + 16 + _i


def _table_body(e_ref, w_ref, b_ref, out_ref):
    prod = lax.dot_general(
        e_ref[...], w_ref[...],
        dimension_numbers=(((1,), (1,)), ((), ())),
        preferred_element_type=jnp.float32,
    )
    out_ref[...] = ((prod + b_ref[...]) * (1.0 / L)).astype(jnp.bfloat16)


def _transform_table(emb_table, W, b):
    return pl.pallas_call(
        _table_body,
        out_shape=jax.ShapeDtypeStruct((VOCAB, D), jnp.bfloat16),
    )(emb_table, W[_PERM], b[_PERM].reshape(1, D))


def _sc_body(table_hbm, idx_hbm, out_hbm, idx_v, rows0, rows1, out_v,
             sem0, sem1):
    wid = lax.axis_index("s") * _NC + lax.axis_index("c")
    base_idx = wid * _IDX_PER_W

    pltpu.sync_copy(idx_hbm.at[pl.ds(base_idx, _IDX_PER_W)], idx_v)

    bufs = (rows0, rows1)
    sems = (sem0, sem1)
    lo_mask = jnp.full((16,), -65536, dtype=jnp.int32)

    def copies(c, k):
        buf, sem = bufs[k], sems[k]
        return [
            pltpu.make_async_copy(
                table_hbm.at[idx_v.at[pl.ds(c * _CHUNK_IDX + j * 128, 128)]],
                buf.at[pl.ds(j * 128, 128)],
                sem,
            )
            for j in range(_STREAMS)
        ]

    def fire(c, k):
        for h in copies(c, k):
            h.start()

    def drain(c, k):
        for h in copies(c, k):
            h.wait()

    def compute(c, k):
        buf = bufs[k]

        def seq_body(s, _):
            base = s * L
            accs = [None] * 4
            for l in range(L):
                for h in range(2):
                    w = plsc.bitcast(buf[base + l, pl.ds(32 * h, 32)],
                                     jnp.int32)
                    lo = plsc.bitcast(w << 16, jnp.float32)
                    hi = plsc.bitcast(w & lo_mask, jnp.float32)
                    if l == 0:
                        accs[2 * h] = lo
                        accs[2 * h + 1] = hi
                    else:
                        accs[2 * h] = accs[2 * h] + lo
                        accs[2 * h + 1] = accs[2 * h + 1] + hi
            for q in range(4):
                out_v[c * _CHUNK_SEQ + s, pl.ds(q * 16, 16)] = accs[q]
            return 0

        lax.fori_loop(0, _CHUNK_SEQ, seq_body, 0, unroll=2)

    fire(0, 0)

    def pair_body(p):
        c0 = p * 2
        c1 = c0 + 1
        fire(c1, 1)
        drain(c0, 0)
        compute(c0, 0)

        @pl.when(c1 + 1 < _N_CHUNKS)
        def _():
            fire(c1 + 1, 0)

        drain(c1, 1)
        compute(c1, 1)

    pl.loop(0, _N_CHUNKS // 2)(pair_body)

    pltpu.sync_copy(out_v, out_hbm.at[pl.ds(wid * _SEQ_PER_W, _SEQ_PER_W)])


@functools.partial(
    pl.kernel,
    out_type=jax.ShapeDtypeStruct((B, D), jnp.float32),
    mesh=plsc.VectorSubcoreMesh(core_axis_name="c", subcore_axis_name="s"),
    compiler_params=pltpu.CompilerParams(use_tc_tiling_on_sc=False,
                                         needs_layout_passes=False),
    scratch_types=[
        pltpu.VMEM((_IDX_PER_W,), jnp.int32),
        pltpu.VMEM((_CHUNK_IDX, D), jnp.bfloat16),
        pltpu.VMEM((_CHUNK_IDX, D), jnp.bfloat16),
        pltpu.VMEM((_SEQ_PER_W, D), jnp.float32),
        pltpu.SemaphoreType.DMA,
        pltpu.SemaphoreType.DMA,
    ],
)
def _sc_gather_pool(table_hbm, idx_hbm, out_hbm, idx_v, rows0, rows1, out_v,
                    sem0, sem1):
    _sc_body(table_hbm, idx_hbm, out_hbm, idx_v, rows0, rows1, out_v,
             sem0, sem1)


@jax.jit
def kernel(input_ids, emb_table, W, b):
    table2 = _transform_table(emb_table, W, b)
    idx = input_ids.astype(jnp.int32).reshape(-1)
    return _sc_gather_pool(table2, idx)
